# baseline (speedup 1.0000x reference)
import functools

import jax
import jax.numpy as jnp
from jax import lax
from jax.experimental import pallas as pl
from jax.experimental.pallas import tpu as pltpu
from jax.experimental.pallas import tpu_sc as plsc

_N_BLK = 1024
_IDX_BITS = 0x7FF


def _knn_block_kernel(ux, uy, uz, kx, ky, kz, gidx_ref, wts_ref):
    n_blk = ux.shape[-1]
    m = kx.shape[-1]
    b = pl.program_id(0)
    d = (ux[0, 0, 0, :][:, None] - kx[0, 0, :][None, :]) ** 2
    d += (uy[0, 0, 0, :][:, None] - ky[0, 0, :][None, :]) ** 2
    d += (uz[0, 0, 0, :][:, None] - kz[0, 0, :][None, :]) ** 2

    di = lax.bitcast_convert_type(d, jnp.int32)
    iota = lax.broadcasted_iota(jnp.int32, (n_blk, m), 1)
    key = lax.bitcast_convert_type((di & ~_IDX_BITS) | iota, jnp.float32)

    c1 = jnp.min(key, axis=1, keepdims=True)
    k2 = jnp.where(key == c1, jnp.inf, key)
    c2 = jnp.min(k2, axis=1, keepdims=True)
    k3 = jnp.where(k2 == c2, jnp.inf, k2)
    c3 = jnp.min(k3, axis=1, keepdims=True)

    ci = lax.bitcast_convert_type(
        jnp.concatenate([c1, c2, c3], axis=1), jnp.int32)
    idx = ci & _IDX_BITS
    dv = lax.bitcast_convert_type(ci & ~_IDX_BITS, jnp.float32)
    r = 1.0 / (dv + 1e-8)
    wn = r / jnp.sum(r, axis=1, keepdims=True)
    wts_ref[0] = jnp.broadcast_to(wn[:, :, None], (n_blk, 3, 16))
    gidx_ref[0] = idx + b * m


def _knn_tc(unknown, known):
    B, n, _ = unknown.shape
    _, m, _ = known.shape
    n_blk = _N_BLK

    ux, uy, uz = (unknown[:, :, i].reshape(B, n // n_blk, 1, n_blk)
                  for i in range(3))
    kx, ky, kz = (known[:, :, i].reshape(B, 1, m) for i in range(3))

    grid = (B, n // n_blk)
    u_spec = pl.BlockSpec((1, 1, 1, n_blk), lambda b, i: (b, i, 0, 0))
    k_spec = pl.BlockSpec((1, 1, m), lambda b, i: (b, 0, 0))
    o_spec = pl.BlockSpec((1, n_blk, 3), lambda b, i: (b, i, 0))
    w_spec = pl.BlockSpec((1, n_blk, 3, 16), lambda b, i: (b, i, 0, 0))

    return pl.pallas_call(
        _knn_block_kernel,
        grid=grid,
        in_specs=[u_spec, u_spec, u_spec, k_spec, k_spec, k_spec],
        out_specs=[o_spec, w_spec],
        out_shape=[jax.ShapeDtypeStruct((B, n, 3), jnp.int32),
                   jax.ShapeDtypeStruct((B, n, 3, 16), jnp.float32)],
        compiler_params=pltpu.CompilerParams(
            dimension_semantics=("parallel", "arbitrary"),
        ),
    )(ux, uy, uz, kx, ky, kz)


_P_CHUNK = 32


def _make_sc_interp(total_pts, C):
    info = plsc.get_sparse_core_info()
    NC, NS = info.num_cores, info.num_subcores
    NW = NC * NS
    pts_per_w = total_pts // NW
    n_chunks = pts_per_w // _P_CHUNK
    mesh = plsc.VectorSubcoreMesh(core_axis_name="c", subcore_axis_name="s")
    nj = C // 16

    @functools.partial(
        pl.kernel, mesh=mesh,
        out_type=jax.ShapeDtypeStruct((total_pts, C), jnp.float32),
        scratch_types=[
            pltpu.VMEM((3 * _P_CHUNK,), jnp.int32),
            pltpu.VMEM((3 * _P_CHUNK, 16), jnp.float32),
            pltpu.VMEM((3 * _P_CHUNK, C), jnp.float32),
            pltpu.VMEM((_P_CHUNK, C), jnp.float32),
            pltpu.SemaphoreType.DMA,
        ],
    )
    def sc_interp(table, gidx, wts, out, idx_v, w_v, rows_v, out_v, sem):
        wid = lax.axis_index("s") * NC + lax.axis_index("c")
        base_pt = wid * pts_per_w

        def chunk_body(c, carry):
            g0 = base_pt + c * _P_CHUNK
            pltpu.sync_copy(gidx.at[pl.ds(3 * g0, 3 * _P_CHUNK)], idx_v)
            pltpu.sync_copy(wts.at[pl.ds(3 * g0, 3 * _P_CHUNK)], w_v)
            pltpu.async_copy(table.at[idx_v], rows_v, sem).wait()

            def pt_body(p, carry2):
                row0 = 3 * p
                sw = pl.ds(0, 16)
                w0 = w_v[row0, sw]
                w1 = w_v[row0 + 1, sw]
                w2 = w_v[row0 + 2, sw]
                for j in range(nj):
                    sl = pl.ds(16 * j, 16)
                    acc = (rows_v[row0, sl] * w0
                           + rows_v[row0 + 1, sl] * w1
                           + rows_v[row0 + 2, sl] * w2)
                    out_v[p, sl] = acc
                return carry2

            lax.fori_loop(0, _P_CHUNK, pt_body, 0)
            pltpu.sync_copy(out_v, out.at[pl.ds(g0, _P_CHUNK)])
            return carry

        lax.fori_loop(0, n_chunks, chunk_body, 0)

    return sc_interp


@jax.jit
def kernel(unknown, known, known_feats):
    B, n, _ = unknown.shape
    _, m, _ = known.shape
    C = known_feats.shape[1]

    gidx, wexp = _knn_tc(unknown, known)
    table = jnp.transpose(known_feats, (0, 2, 1)).reshape(B * m, C)
    out_rows = _make_sc_interp(B * n, C)(
        table, gidx.reshape(-1), wexp.reshape(B * n * 3, 16))
    return jnp.transpose(out_rows.reshape(B, n, C), (0, 2, 1))


# speedup vs baseline: 1.2282x; 1.2282x over previous
import functools

import jax
import jax.numpy as jnp
from jax import lax
from jax.experimental import pallas as pl
from jax.experimental.pallas import tpu as pltpu
from jax.experimental.pallas import tpu_sc as plsc

_N_BLK = 1024
_IDX_BITS = 0x7FF


def _knn_block_kernel(ux, uy, uz, kx, ky, kz, gidx_ref, wts_ref):
    n_blk = ux.shape[-1]
    m = kx.shape[-1]
    b = pl.program_id(0)
    d = (ux[0, 0, 0, :][:, None] - kx[0, 0, :][None, :]) ** 2
    d += (uy[0, 0, 0, :][:, None] - ky[0, 0, :][None, :]) ** 2
    d += (uz[0, 0, 0, :][:, None] - kz[0, 0, :][None, :]) ** 2

    di = lax.bitcast_convert_type(d, jnp.int32)
    iota = lax.broadcasted_iota(jnp.int32, (n_blk, m), 1)
    key = lax.bitcast_convert_type((di & ~_IDX_BITS) | iota, jnp.float32)

    c1 = jnp.min(key, axis=1, keepdims=True)
    k2 = jnp.where(key == c1, jnp.inf, key)
    c2 = jnp.min(k2, axis=1, keepdims=True)
    k3 = jnp.where(k2 == c2, jnp.inf, k2)
    c3 = jnp.min(k3, axis=1, keepdims=True)

    ci = lax.bitcast_convert_type(
        jnp.concatenate([c1, c2, c3], axis=1), jnp.int32)
    idx = ci & _IDX_BITS
    dv = lax.bitcast_convert_type(ci & ~_IDX_BITS, jnp.float32)
    r = 1.0 / (dv + 1e-8)
    wn = r / jnp.sum(r, axis=1, keepdims=True)
    wts_ref[0] = jnp.broadcast_to(wn[:, :, None], (n_blk, 3, 16))
    gidx_ref[0] = idx + b * m


def _knn_tc(unknown, known):
    B, n, _ = unknown.shape
    _, m, _ = known.shape
    n_blk = _N_BLK

    ux, uy, uz = (unknown[:, :, i].reshape(B, n // n_blk, 1, n_blk)
                  for i in range(3))
    kx, ky, kz = (known[:, :, i].reshape(B, 1, m) for i in range(3))

    grid = (B, n // n_blk)
    u_spec = pl.BlockSpec((1, 1, 1, n_blk), lambda b, i: (b, i, 0, 0))
    k_spec = pl.BlockSpec((1, 1, m), lambda b, i: (b, 0, 0))
    o_spec = pl.BlockSpec((1, n_blk, 3), lambda b, i: (b, i, 0))
    w_spec = pl.BlockSpec((1, n_blk, 3, 16), lambda b, i: (b, i, 0, 0))

    return pl.pallas_call(
        _knn_block_kernel,
        grid=grid,
        in_specs=[u_spec, u_spec, u_spec, k_spec, k_spec, k_spec],
        out_specs=[o_spec, w_spec],
        out_shape=[jax.ShapeDtypeStruct((B, n, 3), jnp.int32),
                   jax.ShapeDtypeStruct((B, n, 3, 16), jnp.float32)],
        compiler_params=pltpu.CompilerParams(
            dimension_semantics=("parallel", "arbitrary"),
        ),
    )(ux, uy, uz, kx, ky, kz)


_P_CHUNK = 32


def _make_sc_interp(total_pts, C):
    info = plsc.get_sparse_core_info()
    NC, NS = info.num_cores, info.num_subcores
    NW = NC * NS
    pts_per_w = total_pts // NW
    n_ch = pts_per_w // _P_CHUNK
    mesh = plsc.VectorSubcoreMesh(core_axis_name="c", subcore_axis_name="s")
    nj = C // 16
    I = 3 * _P_CHUNK

    @functools.partial(
        pl.kernel, mesh=mesh,
        out_type=jax.ShapeDtypeStruct((total_pts, C), jnp.float32),
        scratch_types=[
            pltpu.VMEM((n_ch, I), jnp.int32),
            pltpu.VMEM((2, I, 16), jnp.float32),
            pltpu.VMEM((2, I, C), jnp.float32),
            pltpu.VMEM((2, _P_CHUNK, C), jnp.float32),
            pltpu.SemaphoreType.DMA,
            pltpu.SemaphoreType.DMA,
            pltpu.SemaphoreType.DMA,
            pltpu.SemaphoreType.DMA,
            pltpu.SemaphoreType.DMA,
            pltpu.SemaphoreType.DMA,
        ],
    )
    def sc_interp(table, gidx, wts, out, idx_all, w_v, rows_v, out_v,
                  g0s, g1s, w0s, w1s, o0s, o1s):
        wid = lax.axis_index("s") * NC + lax.axis_index("c")
        base_pt = wid * pts_per_w
        base_ch = wid * n_ch
        gsem, wsem, osem = (g0s, g1s), (w0s, w1s), (o0s, o1s)

        pltpu.sync_copy(gidx.at[pl.ds(base_ch, n_ch)], idx_all)

        def issue(c, b):
            pltpu.async_copy(table.at[idx_all.at[c]], rows_v.at[b], gsem[b])
            pltpu.async_copy(wts.at[pl.ds((base_ch + c) * I, I)],
                             w_v.at[b], wsem[b])

        def wait_in(b):
            pltpu.make_async_copy(
                table.at[idx_all.at[0]], rows_v.at[b], gsem[b]).wait()
            pltpu.make_async_copy(
                wts.at[pl.ds(0, I)], w_v.at[b], wsem[b]).wait()

        def wait_out(b):
            pltpu.make_async_copy(
                out_v.at[b], out.at[pl.ds(0, _P_CHUNK)], osem[b]).wait()

        def compute(c, b):
            def pt_body(p, carry):
                row0 = 3 * p
                sw = pl.ds(0, 16)
                w0 = w_v[b, row0, sw]
                w1 = w_v[b, row0 + 1, sw]
                w2 = w_v[b, row0 + 2, sw]
                for j in range(nj):
                    sl = pl.ds(16 * j, 16)
                    acc = (rows_v[b, row0, sl] * w0
                           + rows_v[b, row0 + 1, sl] * w1
                           + rows_v[b, row0 + 2, sl] * w2)
                    out_v[b, p, sl] = acc
                return carry

            lax.fori_loop(0, _P_CHUNK, pt_body, 0)
            pltpu.async_copy(
                out_v.at[b],
                out.at[pl.ds(base_pt + c * _P_CHUNK, _P_CHUNK)], osem[b])

        issue(0, 0)

        def body(i, carry):
            c0 = 2 * i
            issue(c0 + 1, 1)
            wait_in(0)

            @pl.when(i > 0)
            def _():
                wait_out(0)

            compute(c0, 0)

            @pl.when(c0 + 2 < n_ch)
            def _():
                issue(c0 + 2, 0)

            wait_in(1)

            @pl.when(i > 0)
            def _():
                wait_out(1)

            compute(c0 + 1, 1)
            return carry

        lax.fori_loop(0, n_ch // 2, body, 0)
        wait_out(0)
        wait_out(1)

    return sc_interp


@jax.jit
def kernel(unknown, known, known_feats):
    B, n, _ = unknown.shape
    _, m, _ = known.shape
    C = known_feats.shape[1]

    gidx, wexp = _knn_tc(unknown, known)
    table = jnp.transpose(known_feats, (0, 2, 1)).reshape(B * m, C)
    out_rows = _make_sc_interp(B * n, C)(
        table, gidx.reshape(B * n * 3 // 96, 96),
        wexp.reshape(B * n * 3, 16))
    return jnp.transpose(out_rows.reshape(B, n, C), (0, 2, 1))


# speedup vs baseline: 2.8932x; 2.3555x over previous
import functools

import jax
import jax.numpy as jnp
from jax.experimental import pallas as pl
from jax.experimental.pallas import tpu as pltpu

_N_BLK = 2048


def _fp_block_kernel(ux, uy, uz, kx, ky, kz, feats, out_ref):
    d = (ux[0, 0, 0, :][:, None] - kx[0, 0, :][None, :]) ** 2
    d += (uy[0, 0, 0, :][:, None] - ky[0, 0, :][None, :]) ** 2
    d += (uz[0, 0, 0, :][:, None] - kz[0, 0, :][None, :]) ** 2

    v1 = jnp.min(d, axis=1, keepdims=True)
    d2 = jnp.where(d == v1, jnp.inf, d)
    v2 = jnp.min(d2, axis=1, keepdims=True)
    d3 = jnp.where(d2 == v2, jnp.inf, d2)
    v3 = jnp.min(d3, axis=1, keepdims=True)

    w = jnp.where(d <= v3, 1.0 / (d + 1e-8), 0.0)
    norm = (1.0 / (v1 + 1e-8) + 1.0 / (v2 + 1e-8)
            + 1.0 / (v3 + 1e-8))[:, 0]

    out = jax.lax.dot_general(
        feats[0], w,
        dimension_numbers=(((1,), (1,)), ((), ())),
        preferred_element_type=jnp.float32,
    )
    out_ref[0] = out * (1.0 / norm)[None, :]


@jax.jit
def kernel(unknown, known, known_feats):
    B, n, _ = unknown.shape
    _, m, _ = known.shape
    C = known_feats.shape[1]
    n_blk = _N_BLK

    ux, uy, uz = (unknown[:, :, i].reshape(B, n // n_blk, 1, n_blk)
                  for i in range(3))
    kx, ky, kz = (known[:, :, i].reshape(B, 1, m) for i in range(3))

    grid = (B, n // n_blk)
    u_spec = pl.BlockSpec((1, 1, 1, n_blk), lambda b, i: (b, i, 0, 0))
    k_spec = pl.BlockSpec((1, 1, m), lambda b, i: (b, 0, 0))
    f_spec = pl.BlockSpec((1, C, m), lambda b, i: (b, 0, 0))
    out_spec = pl.BlockSpec((1, C, n_blk), lambda b, i: (b, 0, i))

    return pl.pallas_call(
        _fp_block_kernel,
        grid=grid,
        in_specs=[u_spec, u_spec, u_spec, k_spec, k_spec, k_spec, f_spec],
        out_specs=out_spec,
        out_shape=jax.ShapeDtypeStruct((B, C, n), jnp.float32),
        compiler_params=pltpu.CompilerParams(
            dimension_semantics=("parallel", "arbitrary"),
        ),
    )(ux, uy, uz, kx, ky, kz, known_feats)
